# baseline (speedup 1.0000x reference)
import functools

import jax
import jax.numpy as jnp
from jax.experimental import pallas as pl
from jax.experimental.pallas import tpu as pltpu

B = 64
S = 512
D = 2048
K = 8192
KT = 512
ALPHA_BASE = 0.3
MAX_DELTA = 0.5


def _mean_kernel(h_ref, o_ref):
    o_ref[...] = jnp.mean(h_ref[0], axis=0)[None, None, :]


def _lookup_kernel(vraw_ref, a_ref, o_ref, vnorm_ref, best_ref, rmax_ref):
    j = pl.program_id(0)
    nsteps = pl.num_programs(0)

    @pl.when(j == 0)
    def _init():
        v = vraw_ref[...]
        bm = jnp.mean(v, axis=0)
        bv = jnp.mean((v - bm[None, :]) ** 2, axis=0)
        vnorm_ref[...] = (v - bm[None, :]) / jnp.sqrt(bv + 1e-8)[None, :]
        rmax_ref[...] = jnp.full((B, 128), -jnp.inf, jnp.float32)
        best_ref[...] = jnp.zeros((B, D), jnp.float32)

    a = a_ref[...]
    rn = 1.0 / jnp.maximum(
        jnp.sqrt(jnp.sum(a * a, axis=1)), 1e-8)
    vn = vnorm_ref[...]
    cos = jax.lax.dot_general(
        vn, a, (((1,), (1,)), ((), ())),
        preferred_element_type=jnp.float32,
        precision=jax.lax.Precision.HIGHEST)
    cos = cos * rn[None, :]
    tile_max = jnp.max(cos, axis=1)
    tile_arg = jnp.argmax(cos, axis=1)
    run_max = rmax_ref[:, 0]
    improved = tile_max > run_max
    onehot = jnp.where(
        jax.lax.broadcasted_iota(jnp.int32, (B, KT), 1) == tile_arg[:, None],
        rn[None, :], 0.0)
    cand = jax.lax.dot_general(
        onehot, a, (((1,), (0,)), ((), ())),
        preferred_element_type=jnp.float32)
    best_ref[...] = jnp.where(improved[:, None], cand, best_ref[...])
    new_max = jnp.where(improved, tile_max, run_max)
    rmax_ref[...] = jnp.broadcast_to(new_max[:, None], (B, 128))

    @pl.when(j == nsteps - 1)
    def _finish():
        vnorm = vnorm_ref[...]
        score = rmax_ref[:, 0]
        alpha = ALPHA_BASE * (1.0 - score)
        delta = jnp.clip(best_ref[...] - vnorm, -MAX_DELTA, MAX_DELTA)
        v_snapped = vnorm + alpha[:, None] * delta
        o_ref[...] = v_snapped - vraw_ref[...]


def _add_kernel(h_ref, d_ref, o_ref):
    o_ref[...] = h_ref[...] + d_ref[...]


@jax.jit
def kernel(hidden_states, attractors):
    v_raw = pl.pallas_call(
        _mean_kernel,
        grid=(B,),
        in_specs=[pl.BlockSpec((1, S, D), lambda i: (i, 0, 0))],
        out_specs=pl.BlockSpec((1, 1, D), lambda i: (i, 0, 0)),
        out_shape=jax.ShapeDtypeStruct((B, 1, D), jnp.float32),
    )(hidden_states)
    v_raw = v_raw.reshape(B, D)

    v_diff = pl.pallas_call(
        _lookup_kernel,
        grid=(K // KT,),
        in_specs=[
            pl.BlockSpec((B, D), lambda j: (0, 0)),
            pl.BlockSpec((KT, D), lambda j: (j, 0)),
        ],
        out_specs=pl.BlockSpec((B, D), lambda j: (0, 0)),
        out_shape=jax.ShapeDtypeStruct((B, D), jnp.float32),
        scratch_shapes=[
            pltpu.VMEM((B, D), jnp.float32),
            pltpu.VMEM((B, D), jnp.float32),
            pltpu.VMEM((B, 128), jnp.float32),
        ],
    )(v_raw, attractors)

    out = pl.pallas_call(
        _add_kernel,
        grid=(B,),
        in_specs=[
            pl.BlockSpec((1, S, D), lambda i: (i, 0, 0)),
            pl.BlockSpec((1, 1, D), lambda i: (i, 0, 0)),
        ],
        out_specs=pl.BlockSpec((1, S, D), lambda i: (i, 0, 0)),
        out_shape=jax.ShapeDtypeStruct((B, S, D), jnp.float32),
    )(hidden_states, v_diff.reshape(B, 1, D))
    return out


# speedup vs baseline: 1.2013x; 1.2013x over previous
import jax
import jax.numpy as jnp
from jax.experimental import pallas as pl
from jax.experimental.pallas import tpu as pltpu

B = 64
S = 512
D = 2048
K = 8192
KT = 512
NKT = K // KT
P1 = B
P2 = NKT
ALPHA_BASE = 0.3
MAX_DELTA = 0.5


def _fused_kernel(h_ref, a_ref, o_ref, vraw_ref, vnorm_ref, best_ref, rmax_ref,
                  vdiff_ref):
    i = pl.program_id(0)

    @pl.when(i < P1)
    def _phase_mean():
        vraw_ref[pl.ds(i, 1), :] = jnp.mean(h_ref[0], axis=0)[None, :]

    @pl.when(jnp.logical_and(i >= P1, i < P1 + P2))
    def _phase_lookup():
        j = i - P1

        @pl.when(j == 0)
        def _init():
            v = vraw_ref[...]
            bm = jnp.mean(v, axis=0)
            bv = jnp.mean((v - bm[None, :]) ** 2, axis=0)
            vnorm_ref[...] = (v - bm[None, :]) / jnp.sqrt(bv + 1e-8)[None, :]
            rmax_ref[...] = jnp.full((B, 128), -jnp.inf, jnp.float32)
            best_ref[...] = jnp.zeros((B, D), jnp.float32)

        a = a_ref[...]
        rn = 1.0 / jnp.maximum(jnp.sqrt(jnp.sum(a * a, axis=1)), 1e-8)
        vn = vnorm_ref[...]
        cos = jax.lax.dot_general(
            vn, a, (((1,), (1,)), ((), ())),
            preferred_element_type=jnp.float32)
        cos = cos * rn[None, :]
        tile_max = jnp.max(cos, axis=1)
        tile_arg = jnp.argmax(cos, axis=1)
        run_max = rmax_ref[:, 0]
        improved = tile_max > run_max
        onehot = jnp.where(
            jax.lax.broadcasted_iota(jnp.int32, (B, KT), 1) == tile_arg[:, None],
            rn[None, :], 0.0)
        cand = jax.lax.dot_general(
            onehot, a, (((1,), (0,)), ((), ())),
            preferred_element_type=jnp.float32)
        best_ref[...] = jnp.where(improved[:, None], cand, best_ref[...])
        new_max = jnp.where(improved, tile_max, run_max)
        rmax_ref[...] = jnp.broadcast_to(new_max[:, None], (B, 128))

        @pl.when(j == P2 - 1)
        def _finish():
            vnorm = vnorm_ref[...]
            score = rmax_ref[:, 0]
            alpha = ALPHA_BASE * (1.0 - score)
            delta = jnp.clip(best_ref[...] - vnorm, -MAX_DELTA, MAX_DELTA)
            v_snapped = vnorm + alpha[:, None] * delta
            vdiff_ref[...] = v_snapped - vraw_ref[...]

    @pl.when(i >= P1 + P2)
    def _phase_add():
        b = i - (P1 + P2)
        o_ref[...] = h_ref[...] + vdiff_ref[pl.ds(b, 1), :][None, :, :]


def _h_index(i):
    b = jnp.where(i < P1, i, jnp.where(i < P1 + P2, P1 - 1, i - (P1 + P2)))
    return (b, 0, 0)


def _a_index(i):
    j = jnp.clip(i - P1, 0, P2 - 1)
    return (j, 0)


def _o_index(i):
    b = jnp.where(i < P1 + P2, 0, i - (P1 + P2))
    return (b, 0, 0)


@jax.jit
def kernel(hidden_states, attractors):
    return pl.pallas_call(
        _fused_kernel,
        grid=(P1 + P2 + B,),
        in_specs=[
            pl.BlockSpec((1, S, D), _h_index),
            pl.BlockSpec((KT, D), _a_index),
        ],
        out_specs=pl.BlockSpec((1, S, D), _o_index),
        out_shape=jax.ShapeDtypeStruct((B, S, D), jnp.float32),
        scratch_shapes=[
            pltpu.VMEM((B, D), jnp.float32),
            pltpu.VMEM((B, D), jnp.float32),
            pltpu.VMEM((B, D), jnp.float32),
            pltpu.VMEM((B, 128), jnp.float32),
            pltpu.VMEM((B, D), jnp.float32),
        ],
    )(hidden_states, attractors)


# speedup vs baseline: 1.2529x; 1.0430x over previous
import jax
import jax.numpy as jnp
from jax.experimental import pallas as pl
from jax.experimental.pallas import tpu as pltpu

B = 64
S = 512
D = 2048
K = 8192
KT = 1024
NKT = K // KT
RB = 2
P1 = B // RB
P2 = NKT
ALPHA_BASE = 0.3
MAX_DELTA = 0.5


def _fused_kernel(h_ref, a_ref, o_ref, vraw_ref, vnorm_ref, best_ref, rmax_ref,
                  vdiff_ref):
    i = pl.program_id(0)

    @pl.when(i < P1)
    def _phase_mean():
        m = jnp.mean(h_ref[...], axis=1)
        for r in range(RB):
            vraw_ref[pl.ds(i * RB + r, 1), :] = m[r][None, :]

    @pl.when(jnp.logical_and(i >= P1, i < P1 + P2))
    def _phase_lookup():
        j = i - P1

        @pl.when(j == 0)
        def _init():
            v = vraw_ref[...]
            bm = jnp.mean(v, axis=0)
            bv = jnp.mean((v - bm[None, :]) ** 2, axis=0)
            vnorm_ref[...] = (v - bm[None, :]) / jnp.sqrt(bv + 1e-8)[None, :]
            rmax_ref[...] = jnp.full((B, 128), -jnp.inf, jnp.float32)
            best_ref[...] = jnp.zeros((B, D), jnp.float32)

        a = a_ref[...]
        rn = 1.0 / jnp.maximum(jnp.sqrt(jnp.sum(a * a, axis=1)), 1e-8)
        vn = vnorm_ref[...]
        cos = jax.lax.dot_general(
            vn, a, (((1,), (1,)), ((), ())),
            preferred_element_type=jnp.float32)
        cos = cos * rn[None, :]
        tile_max = jnp.max(cos, axis=1)
        tile_arg = jnp.argmax(cos, axis=1)
        run_max = rmax_ref[:, 0]
        improved = tile_max > run_max
        onehot = jnp.where(
            jax.lax.broadcasted_iota(jnp.int32, (B, KT), 1) == tile_arg[:, None],
            rn[None, :], 0.0)
        cand = jax.lax.dot_general(
            onehot, a, (((1,), (0,)), ((), ())),
            preferred_element_type=jnp.float32)
        best_ref[...] = jnp.where(improved[:, None], cand, best_ref[...])
        new_max = jnp.where(improved, tile_max, run_max)
        rmax_ref[...] = jnp.broadcast_to(new_max[:, None], (B, 128))

        @pl.when(j == P2 - 1)
        def _finish():
            vnorm = vnorm_ref[...]
            score = rmax_ref[:, 0]
            alpha = ALPHA_BASE * (1.0 - score)
            delta = jnp.clip(best_ref[...] - vnorm, -MAX_DELTA, MAX_DELTA)
            v_snapped = vnorm + alpha[:, None] * delta
            vdiff_ref[...] = v_snapped - vraw_ref[...]

    @pl.when(i >= P1 + P2)
    def _phase_add():
        b = i - (P1 + P2)
        rows = [vdiff_ref[pl.ds(b * RB + r, 1), :] for r in range(RB)]
        o_ref[...] = h_ref[...] + jnp.concatenate(rows, axis=0)[:, None, :]


def _h_index(i):
    b = jnp.where(i < P1, i, jnp.where(i < P1 + P2, P1 - 1, i - (P1 + P2)))
    return (b, 0, 0)


def _a_index(i):
    j = jnp.clip(i - P1, 0, P2 - 1)
    return (j, 0)


def _o_index(i):
    b = jnp.where(i < P1 + P2, 0, i - (P1 + P2))
    return (b, 0, 0)


@jax.jit
def kernel(hidden_states, attractors):
    return pl.pallas_call(
        _fused_kernel,
        grid=(P1 + P2 + P1,),
        in_specs=[
            pl.BlockSpec((RB, S, D), _h_index),
            pl.BlockSpec((KT, D), _a_index),
        ],
        out_specs=pl.BlockSpec((RB, S, D), _o_index),
        out_shape=jax.ShapeDtypeStruct((B, S, D), jnp.float32),
        scratch_shapes=[
            pltpu.VMEM((B, D), jnp.float32),
            pltpu.VMEM((B, D), jnp.float32),
            pltpu.VMEM((B, D), jnp.float32),
            pltpu.VMEM((B, 128), jnp.float32),
            pltpu.VMEM((B, D), jnp.float32),
        ],
    )(hidden_states, attractors)
